# baseline (speedup 1.0000x reference)
import functools

import jax
import jax.numpy as jnp
from jax import lax
from jax.experimental import pallas as pl
from jax.experimental.pallas import tpu as pltpu
from jax.experimental.pallas import tpu_sc as plsc

NUM_ROWS = 1000000
BATCH = 16384
D = 30
ROWW = 128
NRW = NUM_ROWS * D // ROWW
MAXR = NRW - 1

_info = plsc.get_sparse_core_info()
NC = _info.num_cores
NS = _info.num_subcores
L = _info.num_lanes
NW = NC * NS
BPW = BATCH // NW
CH = 128
NCHUNK = BPW // CH

_mesh = plsc.VectorSubcoreMesh(core_axis_name="c", subcore_axis_name="s")


@functools.partial(
    pl.kernel,
    mesh=_mesh,
    out_type=jax.ShapeDtypeStruct((BATCH,), jnp.float32),
    compiler_params=pltpu.CompilerParams(
        use_tc_tiling_on_sc=False,
        needs_layout_passes=False,
    ),
    scratch_types=[
        pltpu.VMEM((BPW,), jnp.int32),
        pltpu.VMEM((BPW,), jnp.int32),
        pltpu.VMEM((NCHUNK, CH), jnp.int32),
        pltpu.VMEM((NCHUNK, CH), jnp.int32),
        pltpu.VMEM((NCHUNK, CH), jnp.int32),
        pltpu.VMEM((NCHUNK, CH), jnp.int32),
        pltpu.VMEM((2 * CH, ROWW), jnp.float32),
        pltpu.VMEM((2 * CH, ROWW), jnp.float32),
        pltpu.VMEM((BPW,), jnp.float32),
        pltpu.SemaphoreType.DMA,
    ],
)
def _recommender_sc(uraw_hbm, iraw_hbm, w_hbm, x_hbm, out_hbm,
                    uraw_v, iraw_v, ua_v, ub_v, xa_v, xb_v,
                    wbuf, xbuf, out_v, sem):
    wid = lax.axis_index("s") * NC + lax.axis_index("c")
    base = wid * BPW

    pltpu.sync_copy(uraw_hbm.at[wid], uraw_v)
    pltpu.sync_copy(iraw_hbm.at[wid], iraw_v)

    for j in range(NCHUNK):
        for k in range(CH // L):
            sl = pl.ds(j * CH + k * L, L)
            dsl = pl.ds(k * L, L)
            for raw_v, a_v, b_v in ((uraw_v, ua_v, ub_v),
                                    (iraw_v, xa_v, xb_v)):
                t = raw_v[sl] * D
                a = t >> 7
                a_v[j, dsl] = a
                b_v[j, dsl] = jnp.minimum(a + 1, MAXR)

    lane = lax.iota(jnp.int32, L)

    for j in range(NCHUNK):
        copies = [
            pltpu.async_copy(w_hbm.at[ua_v.at[j]], wbuf.at[pl.ds(0, CH)], sem),
            pltpu.async_copy(w_hbm.at[ub_v.at[j]], wbuf.at[pl.ds(CH, CH)], sem),
            pltpu.async_copy(x_hbm.at[xa_v.at[j]], xbuf.at[pl.ds(0, CH)], sem),
            pltpu.async_copy(x_hbm.at[xb_v.at[j]], xbuf.at[pl.ds(CH, CH)], sem),
        ]
        for c in copies:
            c.wait()

        def group_body(g, carry):
            gs = pl.ds(j * CH + g * L, L)
            rows = g * L + lane
            uo = (uraw_v[gs] * D) & (ROWW - 1)
            io = (iraw_v[gs] * D) & (ROWW - 1)
            acc = jnp.zeros((L,), jnp.float32)
            for d in range(D):
                uw = uo + d
                iw = io + d
                wv = plsc.load_gather(
                    wbuf, [rows + ((uw >> 7) << 7), uw & (ROWW - 1)])
                xv = plsc.load_gather(
                    xbuf, [rows + ((iw >> 7) << 7), iw & (ROWW - 1)])
                acc = acc + wv * xv
            out_v[gs] = acc
            return carry

        lax.fori_loop(0, CH // L, group_body, 0)

    pltpu.sync_copy(out_v, out_hbm.at[pl.ds(base, BPW)])


def kernel(x, W, X):
    uraw = x[:, 0].astype(jnp.int32).reshape(NW, BPW)
    iraw = x[:, 1].astype(jnp.int32).reshape(NW, BPW)
    w128 = W.reshape(NRW, ROWW)
    x128 = X.reshape(NRW, ROWW)
    return _recommender_sc(uraw, iraw, w128, x128)
